# Initial kernel scaffold; baseline (speedup 1.0000x reference)
import jax
import jax.numpy as jnp
from jax.experimental import pallas as pl


def kernel(input, offsets, weight):
    raise NotImplementedError("write your pallas kernel here")


# speedup vs baseline: 108.5980x; 108.5980x over previous
import functools

import jax
import jax.numpy as jnp
from jax import lax
from jax.experimental import pallas as pl
from jax.experimental.pallas import tpu as pltpu
from jax.experimental.pallas import tpu_sc as plsc

NUM_EMB = 100000
DIM = 64
N_IDX = 204800
N_BAGS = 4096

NC = 2
NS = 16
LANES = 16
NW = NC * NS
PER_W = N_IDX // NW
CHUNK = 128
NCHUNK = PER_W // CHUNK
BAGS_PER_S = N_BAGS // NS


def _sc_body(input2d, offsets, weight, out, idx_v, seg_v, offs_v, rows_v,
             zbuf, acc, sem):
    c = lax.axis_index("c")
    s = lax.axis_index("s")
    wid = c * NS + s

    z16 = jnp.zeros((LANES,), jnp.float32)

    def _zero(i, _):
        for col in range(DIM // LANES):
            zbuf[i, pl.ds(col * LANES, LANES)] = z16
        return 0

    lax.fori_loop(0, BAGS_PER_S, _zero, 0)
    pltpu.sync_copy(zbuf, acc.at[pl.ds(s * BAGS_PER_S, BAGS_PER_S)])

    pltpu.sync_copy(input2d.at[wid], idx_v)
    pltpu.sync_copy(offsets, offs_v)

    base = wid * PER_W
    lane = lax.iota(jnp.int32, LANES)

    def _segs(j, _):
        for k in range(CHUNK // LANES):
            p = base + j * CHUNK + k * LANES + lane
            lo = jnp.zeros((LANES,), jnp.int32)
            step = N_BAGS // 2
            while step >= 1:
                cand = lo + step
                inb = cand <= N_BAGS - 1
                candc = jnp.where(inb, cand, N_BAGS - 1)
                vals = plsc.load_gather(offs_v, (candc,))
                ok = jnp.logical_and(inb, vals <= p)
                lo = jnp.where(ok, cand, lo)
                step //= 2
            seg_v[j, pl.ds(k * LANES, LANES)] = lo
        return 0

    lax.fori_loop(0, NCHUNK, _segs, 0)

    plsc.subcore_barrier()

    def _accum(j, _):
        pltpu.async_copy(weight.at[idx_v.at[j]], rows_v, sem).wait()
        pltpu.sync_copy(rows_v, acc.at[seg_v.at[j]], add=True)
        return 0

    lax.fori_loop(0, NCHUNK, _accum, 0)

    plsc.subcore_barrier()

    sl = pl.ds(s * BAGS_PER_S, BAGS_PER_S)
    pltpu.sync_copy(acc.at[sl], out.at[c, sl])


_sc_embedding_sum = functools.partial(
    pl.kernel,
    out_type=jax.ShapeDtypeStruct((NC, N_BAGS, DIM), jnp.float32),
    mesh=plsc.VectorSubcoreMesh(
        core_axis_name="c", subcore_axis_name="s", num_cores=NC,
        num_subcores=NS),
    compiler_params=pltpu.CompilerParams(
        needs_layout_passes=False, use_tc_tiling_on_sc=False),
    scratch_types=[
        pltpu.VMEM((NCHUNK, CHUNK), jnp.int32),
        pltpu.VMEM((NCHUNK, CHUNK), jnp.int32),
        pltpu.VMEM((N_BAGS,), jnp.int32),
        pltpu.VMEM((CHUNK, DIM), jnp.float32),
        pltpu.VMEM((BAGS_PER_S, DIM), jnp.float32),
        pltpu.VMEM_SHARED((N_BAGS, DIM), jnp.float32),
        pltpu.SemaphoreType.DMA,
    ],
)(_sc_body)


def _tc_add(p_ref, o_ref):
    o_ref[...] = p_ref[0] + p_ref[1]


_combine = pl.pallas_call(
    _tc_add,
    out_shape=jax.ShapeDtypeStruct((N_BAGS, DIM), jnp.float32),
)


@jax.jit
def kernel(input, offsets, weight):
    input2d = input.reshape(NW, NCHUNK, CHUNK)
    partials = _sc_embedding_sum(input2d, offsets, weight)
    return _combine(partials)


# speedup vs baseline: 141.6620x; 1.3045x over previous
import functools

import jax
import jax.numpy as jnp
from jax import lax
from jax.experimental import pallas as pl
from jax.experimental.pallas import tpu as pltpu
from jax.experimental.pallas import tpu_sc as plsc

NUM_EMB = 100000
DIM = 64
N_IDX = 204800
N_BAGS = 4096

NC = 2
NS = 16
LANES = 16
NW = NC * NS
PER_W = N_IDX // NW
CHUNK = 128
NCHUNK = PER_W // CHUNK
BAGS_PER_S = N_BAGS // NS


def _sc_body(input2d, offsets, weight, out, idx_v, seg_v, offs_v, rows_v,
             zbuf, acc, sem):
    c = lax.axis_index("c")
    s = lax.axis_index("s")
    wid = c * NS + s

    z16 = jnp.zeros((LANES,), jnp.float32)

    def _zero(i, _):
        for col in range(DIM // LANES):
            zbuf[i, pl.ds(col * LANES, LANES)] = z16
        return 0

    lax.fori_loop(0, BAGS_PER_S, _zero, 0)
    pltpu.sync_copy(zbuf, acc.at[pl.ds(s * BAGS_PER_S, BAGS_PER_S)])

    pltpu.sync_copy(input2d.at[wid], idx_v)
    pltpu.sync_copy(offsets, offs_v)

    plsc.subcore_barrier()

    base = wid * PER_W
    lane = lax.iota(jnp.int32, LANES)

    pltpu.async_copy(weight.at[idx_v.at[0]], rows_v.at[0], sem)

    def _accum(j, _):
        nxt = j + 1

        @pl.when(nxt < NCHUNK)
        def _fire():
            pltpu.async_copy(weight.at[idx_v.at[nxt]], rows_v.at[nxt % 2],
                             sem)

        for k in range(CHUNK // LANES):
            p = base + j * CHUNK + k * LANES + lane
            lo = jnp.zeros((LANES,), jnp.int32)
            step = N_BAGS // 2
            while step >= 1:
                cand = lo + step
                inb = cand <= N_BAGS - 1
                candc = jnp.where(inb, cand, N_BAGS - 1)
                vals = plsc.load_gather(offs_v, (candc,))
                ok = jnp.logical_and(inb, vals <= p)
                lo = jnp.where(ok, cand, lo)
                step //= 2
            seg_v[j, pl.ds(k * LANES, LANES)] = lo

        pltpu.make_async_copy(weight.at[idx_v.at[j]], rows_v.at[j % 2],
                              sem).wait()
        pltpu.sync_copy(rows_v.at[j % 2], acc.at[seg_v.at[j]], add=True)
        return 0

    lax.fori_loop(0, NCHUNK, _accum, 0)

    plsc.subcore_barrier()

    sl = pl.ds(s * BAGS_PER_S, BAGS_PER_S)
    pltpu.sync_copy(acc.at[sl], out.at[c, sl])


_sc_embedding_sum = functools.partial(
    pl.kernel,
    out_type=jax.ShapeDtypeStruct((NC, N_BAGS, DIM), jnp.float32),
    mesh=plsc.VectorSubcoreMesh(
        core_axis_name="c", subcore_axis_name="s", num_cores=NC,
        num_subcores=NS),
    compiler_params=pltpu.CompilerParams(
        needs_layout_passes=False, use_tc_tiling_on_sc=False),
    scratch_types=[
        pltpu.VMEM((NCHUNK, CHUNK), jnp.int32),
        pltpu.VMEM((NCHUNK, CHUNK), jnp.int32),
        pltpu.VMEM((N_BAGS,), jnp.int32),
        pltpu.VMEM((2, CHUNK, DIM), jnp.float32),
        pltpu.VMEM((BAGS_PER_S, DIM), jnp.float32),
        pltpu.VMEM_SHARED((N_BAGS, DIM), jnp.float32),
        pltpu.SemaphoreType.DMA,
    ],
)(_sc_body)


def _tc_add(p_ref, o_ref):
    o_ref[...] = p_ref[0] + p_ref[1]


_combine = pl.pallas_call(
    _tc_add,
    out_shape=jax.ShapeDtypeStruct((N_BAGS, DIM), jnp.float32),
)


@jax.jit
def kernel(input, offsets, weight):
    input2d = input.reshape(NW, NCHUNK, CHUNK)
    partials = _sc_embedding_sum(input2d, offsets, weight)
    return _combine(partials)
